# Initial kernel scaffold; baseline (speedup 1.0000x reference)
import jax
import jax.numpy as jnp
from jax.experimental import pallas as pl


def kernel(wft_ics, wft_vals, bft_ics, bft_vals, stm, W_ft, b_ft, W1, b1, W2, b2, Wo, bo):
    raise NotImplementedError("write your pallas kernel here")


# speedup vs baseline: 1.3975x; 1.3975x over previous
import functools

import jax
import jax.numpy as jnp
from jax import lax
from jax.experimental import pallas as pl
from jax.experimental.pallas import tpu as pltpu
from jax.experimental.pallas import tpu_sc as plsc

N_FTS = 100000
D = 256
B = 16384
L = 32

NC = 2
NS = 16
NW = NC * NS
LANES = 16

SAMPLES = 2 * B
SPW = SAMPLES // NW
CHUNK = 4
NCHUNKS = SPW // CHUNK


def _ft_body(ics_hbm, vals_hbm, table_hbm, out_hbm, idx_v, vals_v, rows_v, acc_v, sem):
    wid = lax.axis_index("s") * NC + lax.axis_index("c")
    base = wid * SPW

    def chunk_body(g, carry):
        s0 = base + g * CHUNK
        pltpu.sync_copy(ics_hbm.at[pl.ds(s0 * L, CHUNK * L)], idx_v)
        pltpu.sync_copy(vals_hbm.at[pl.ds(s0 * L, CHUNK * L)], vals_v)
        pltpu.async_copy(table_hbm.at[idx_v], rows_v, sem).wait()

        def sample_body(i, carry2):
            v0 = vals_v[pl.ds(i * L, LANES)]
            v1 = vals_v[pl.ds(i * L + LANES, LANES)]
            accs = [jnp.zeros((LANES,), jnp.float32) for _ in range(D // LANES)]
            for l in range(L):
                vv = v0 if l < LANES else v1
                val = lax.index_in_dim(vv, l % LANES, 0, keepdims=False)
                r = i * L + l
                for j in range(D // LANES):
                    accs[j] = accs[j] + rows_v[r, pl.ds(j * LANES, LANES)] * val
            for j in range(D // LANES):
                acc_v[i, pl.ds(j * LANES, LANES)] = accs[j]
            return carry2

        lax.fori_loop(0, CHUNK, sample_body, 0)
        pltpu.sync_copy(acc_v, out_hbm.at[pl.ds(s0, CHUNK)])
        return carry

    lax.fori_loop(0, NCHUNKS, chunk_body, 0)


def _feature_transform(ics_flat, vals_flat, table):
    mesh = plsc.VectorSubcoreMesh(core_axis_name="c", subcore_axis_name="s")
    return pl.kernel(
        _ft_body,
        out_type=jax.ShapeDtypeStruct((SAMPLES, D), jnp.float32),
        mesh=mesh,
        scratch_types=[
            pltpu.VMEM((CHUNK * L,), jnp.int32),
            pltpu.VMEM((CHUNK * L,), jnp.float32),
            pltpu.VMEM((CHUNK * L, D), jnp.float32),
            pltpu.VMEM((CHUNK, D), jnp.float32),
            pltpu.SemaphoreType.DMA,
        ],
        name="nnue_feature_transform",
    )(ics_flat, vals_flat, table)


def _mlp_body(wf_ref, bf_ref, s_ref, bft_ref, W1_ref, b1_ref, W2_ref, b2_ref,
              Wo_ref, bo_ref, o_ref):
    bft = bft_ref[...]
    wf = wf_ref[...] + bft
    bf = bf_ref[...] + bft
    s = s_ref[...]
    x1 = jnp.clip((1.0 - s) * wf + s * bf, 0.0, 1.0)
    x2 = jnp.clip((1.0 - s) * bf + s * wf, 0.0, 1.0)
    dn = (((1,), (1,)), ((), ()))
    h = lax.dot_general(x1, W1_ref[:, :D], dn, preferred_element_type=jnp.float32)
    h += lax.dot_general(x2, W1_ref[:, D:], dn, preferred_element_type=jnp.float32)
    h = jnp.clip(h + b1_ref[...], 0.0, 1.0)
    h = lax.dot_general(h, W2_ref[...], dn, preferred_element_type=jnp.float32)
    h = jnp.clip(h + b2_ref[...], 0.0, 1.0)
    o_ref[...] = jnp.sum(h * Wo_ref[...], axis=1, keepdims=True) + bo_ref[...]


def _mlp_head(wfts, bfts, stm, b_ft, W1, b1, W2, b2, Wo, bo):
    BB = 2048
    grid = (B // BB,)
    return pl.pallas_call(
        _mlp_body,
        grid=grid,
        in_specs=[
            pl.BlockSpec((BB, D), lambda i: (i, 0)),
            pl.BlockSpec((BB, D), lambda i: (i, 0)),
            pl.BlockSpec((BB, 1), lambda i: (i, 0)),
            pl.BlockSpec((1, D), lambda i: (0, 0)),
            pl.BlockSpec((32, 2 * D), lambda i: (0, 0)),
            pl.BlockSpec((1, 32), lambda i: (0, 0)),
            pl.BlockSpec((32, 32), lambda i: (0, 0)),
            pl.BlockSpec((1, 32), lambda i: (0, 0)),
            pl.BlockSpec((1, 32), lambda i: (0, 0)),
            pl.BlockSpec((1, 1), lambda i: (0, 0)),
        ],
        out_specs=pl.BlockSpec((BB, 1), lambda i: (i, 0)),
        out_shape=jax.ShapeDtypeStruct((B, 1), jnp.float32),
    )(wfts, bfts, stm, b_ft, W1, b1, W2, b2, Wo, bo)


def kernel(wft_ics, wft_vals, bft_ics, bft_vals, stm, W_ft, b_ft, W1, b1, W2, b2, Wo, bo):
    ics_flat = jnp.concatenate([wft_ics, bft_ics]).reshape(-1)
    vals_flat = jnp.concatenate([wft_vals, bft_vals]).reshape(-1)
    fts = _feature_transform(ics_flat, vals_flat, W_ft)
    wfts = fts[:B]
    bfts = fts[B:]
    return _mlp_head(
        wfts, bfts, stm,
        b_ft.reshape(1, D),
        W1, b1.reshape(1, 32), W2, b2.reshape(1, 32),
        Wo.reshape(1, 32), bo.reshape(1, 1),
    )


# speedup vs baseline: 2.3180x; 1.6587x over previous
import functools

import jax
import jax.numpy as jnp
from jax import lax
from jax.experimental import pallas as pl
from jax.experimental.pallas import tpu as pltpu
from jax.experimental.pallas import tpu_sc as plsc

N_FTS = 100000
D = 256
B = 16384
L = 32

NC = 2
NS = 16
NW = NC * NS
LANES = 16

SAMPLES = 2 * B
SPW = SAMPLES // NW
CHUNK = 4
SS = 512
SC_CHUNKS = SS // CHUNK


def _ft_body(ics_hbm, vals_hbm, table_hbm, out_hbm, idxs_v, vals_v,
             rows_a, rows_b, acc_a, acc_b, sem_a, sem_b):
    wid = lax.axis_index("s") * NC + lax.axis_index("c")
    base = wid * SPW

    def start_gather(g, rows_v, sem):
        pltpu.async_copy(
            table_hbm.at[idxs_v.at[pl.ds(g * CHUNK * L, CHUNK * L)]], rows_v, sem
        )

    def wait_gather(rows_v, sem):
        pltpu.make_async_copy(
            table_hbm.at[pl.ds(0, CHUNK * L)], rows_v, sem
        ).wait()

    def compute_chunk(rows_v, acc_v, g, s0):
        def sample_body(i, carry2):
            r0 = (g * CHUNK + i) * L
            v0 = vals_v[pl.ds(r0, LANES)]
            v1 = vals_v[pl.ds(r0 + LANES, LANES)]
            accs = [jnp.zeros((LANES,), jnp.float32) for _ in range(D // LANES)]
            for l in range(L):
                vv = v0 if l < LANES else v1
                val = lax.index_in_dim(vv, l % LANES, 0, keepdims=False)
                r = i * L + l
                for j in range(D // LANES):
                    accs[j] = accs[j] + rows_v[r, pl.ds(j * LANES, LANES)] * val
            for j in range(D // LANES):
                acc_v[i, pl.ds(j * LANES, LANES)] = accs[j]
            return carry2

        lax.fori_loop(0, CHUNK, sample_body, 0)
        pltpu.sync_copy(acc_v, out_hbm.at[pl.ds(s0 + g * CHUNK, CHUNK)])

    def super_body(sidx, carry):
        s0 = base + sidx * SS
        pltpu.sync_copy(ics_hbm.at[pl.ds(s0 * L, SS * L)], idxs_v)
        pltpu.sync_copy(vals_hbm.at[pl.ds(s0 * L, SS * L)], vals_v)
        start_gather(0, rows_a, sem_a)

        def pair_body(h, carry2):
            g0 = 2 * h
            start_gather(g0 + 1, rows_b, sem_b)
            wait_gather(rows_a, sem_a)
            compute_chunk(rows_a, acc_a, g0, s0)

            @pl.when(h < SC_CHUNKS // 2 - 1)
            def _():
                start_gather(g0 + 2, rows_a, sem_a)

            wait_gather(rows_b, sem_b)
            compute_chunk(rows_b, acc_b, g0 + 1, s0)
            return carry2

        lax.fori_loop(0, SC_CHUNKS // 2, pair_body, 0)
        return carry

    lax.fori_loop(0, SPW // SS, super_body, 0)


def _feature_transform(ics_flat, vals_flat, table):
    mesh = plsc.VectorSubcoreMesh(core_axis_name="c", subcore_axis_name="s")
    return pl.kernel(
        _ft_body,
        out_type=jax.ShapeDtypeStruct((SAMPLES, D), jnp.float32),
        mesh=mesh,
        scratch_types=[
            pltpu.VMEM((SS * L,), jnp.int32),
            pltpu.VMEM((SS * L,), jnp.float32),
            pltpu.VMEM((CHUNK * L, D), jnp.float32),
            pltpu.VMEM((CHUNK * L, D), jnp.float32),
            pltpu.VMEM((CHUNK, D), jnp.float32),
            pltpu.VMEM((CHUNK, D), jnp.float32),
            pltpu.SemaphoreType.DMA,
            pltpu.SemaphoreType.DMA,
        ],
        name="nnue_feature_transform",
    )(ics_flat, vals_flat, table)


def _mlp_body(wf_ref, bf_ref, s_ref, bft_ref, W1_ref, b1_ref, W2_ref, b2_ref,
              Wo_ref, bo_ref, o_ref):
    bft = bft_ref[...]
    wf = wf_ref[...] + bft
    bf = bf_ref[...] + bft
    s = s_ref[...]
    x1 = jnp.clip((1.0 - s) * wf + s * bf, 0.0, 1.0)
    x2 = jnp.clip((1.0 - s) * bf + s * wf, 0.0, 1.0)
    dn = (((1,), (1,)), ((), ()))
    h = lax.dot_general(x1, W1_ref[:, :D], dn, preferred_element_type=jnp.float32)
    h += lax.dot_general(x2, W1_ref[:, D:], dn, preferred_element_type=jnp.float32)
    h = jnp.clip(h + b1_ref[...], 0.0, 1.0)
    h = lax.dot_general(h, W2_ref[...], dn, preferred_element_type=jnp.float32)
    h = jnp.clip(h + b2_ref[...], 0.0, 1.0)
    o_ref[...] = jnp.sum(h * Wo_ref[...], axis=1, keepdims=True) + bo_ref[...]


def _mlp_head(wfts, bfts, stm, b_ft, W1, b1, W2, b2, Wo, bo):
    BB = 2048
    grid = (B // BB,)
    return pl.pallas_call(
        _mlp_body,
        grid=grid,
        in_specs=[
            pl.BlockSpec((BB, D), lambda i: (i, 0)),
            pl.BlockSpec((BB, D), lambda i: (i, 0)),
            pl.BlockSpec((BB, 1), lambda i: (i, 0)),
            pl.BlockSpec((1, D), lambda i: (0, 0)),
            pl.BlockSpec((32, 2 * D), lambda i: (0, 0)),
            pl.BlockSpec((1, 32), lambda i: (0, 0)),
            pl.BlockSpec((32, 32), lambda i: (0, 0)),
            pl.BlockSpec((1, 32), lambda i: (0, 0)),
            pl.BlockSpec((1, 32), lambda i: (0, 0)),
            pl.BlockSpec((1, 1), lambda i: (0, 0)),
        ],
        out_specs=pl.BlockSpec((BB, 1), lambda i: (i, 0)),
        out_shape=jax.ShapeDtypeStruct((B, 1), jnp.float32),
    )(wfts, bfts, stm, b_ft, W1, b1, W2, b2, Wo, bo)


def kernel(wft_ics, wft_vals, bft_ics, bft_vals, stm, W_ft, b_ft, W1, b1, W2, b2, Wo, bo):
    ics_flat = jnp.concatenate([wft_ics, bft_ics]).reshape(-1)
    vals_flat = jnp.concatenate([wft_vals, bft_vals]).reshape(-1)
    fts = _feature_transform(ics_flat, vals_flat, W_ft)
    wfts = fts[:B]
    bfts = fts[B:]
    return _mlp_head(
        wfts, bfts, stm,
        b_ft.reshape(1, D),
        W1, b1.reshape(1, 32), W2, b2.reshape(1, 32),
        Wo.reshape(1, 32), bo.reshape(1, 1),
    )


# speedup vs baseline: 2.3859x; 1.0293x over previous
import functools

import jax
import jax.numpy as jnp
from jax import lax
from jax.experimental import pallas as pl
from jax.experimental.pallas import tpu as pltpu
from jax.experimental.pallas import tpu_sc as plsc

N_FTS = 100000
D = 256
B = 16384
L = 32

NC = 2
NS = 16
NW = NC * NS
LANES = 16

SAMPLES = 2 * B
SPW = SAMPLES // NW
CHUNK = 4
SS = 512
SC_CHUNKS = SS // CHUNK


def _ft_body(ics_hbm, vals_hbm, table_hbm, out_hbm, idxs_v, vals_v,
             rows_a, rows_b, acc_a, acc_b, sem_a, sem_b, osem_a, osem_b):
    wid = lax.axis_index("s") * NC + lax.axis_index("c")
    base = wid * SPW

    def start_gather(g, rows_v, sem):
        pltpu.async_copy(
            table_hbm.at[idxs_v.at[pl.ds(g * CHUNK * L, CHUNK * L)]], rows_v, sem
        )

    def wait_gather(rows_v, sem):
        pltpu.make_async_copy(
            table_hbm.at[pl.ds(0, CHUNK * L)], rows_v, sem
        ).wait()

    def compute_chunk(rows_v, acc_v, g, s0, osem):
        def sample_body(i, carry2):
            r0 = (g * CHUNK + i) * L
            v0 = vals_v[pl.ds(r0, LANES)]
            v1 = vals_v[pl.ds(r0 + LANES, LANES)]
            accs = [jnp.zeros((LANES,), jnp.float32) for _ in range(D // LANES)]
            for l in range(L):
                vv = v0 if l < LANES else v1
                val = lax.index_in_dim(vv, l % LANES, 0, keepdims=False)
                r = i * L + l
                for j in range(D // LANES):
                    accs[j] = accs[j] + rows_v[r, pl.ds(j * LANES, LANES)] * val
            for j in range(D // LANES):
                acc_v[i, pl.ds(j * LANES, LANES)] = accs[j]
            return carry2

        lax.fori_loop(0, CHUNK, sample_body, 0)
        pltpu.async_copy(acc_v, out_hbm.at[pl.ds(s0 + g * CHUNK, CHUNK)], osem)

    def wait_out(acc_v, osem):
        pltpu.make_async_copy(acc_v, out_hbm.at[pl.ds(0, CHUNK)], osem).wait()

    def super_body(sidx, carry):
        s0 = base + sidx * SS
        pltpu.sync_copy(ics_hbm.at[pl.ds(s0 * L, SS * L)], idxs_v)
        pltpu.sync_copy(vals_hbm.at[pl.ds(s0 * L, SS * L)], vals_v)
        start_gather(0, rows_a, sem_a)

        def pair_body(h, carry2):
            g0 = 2 * h
            start_gather(g0 + 1, rows_b, sem_b)
            wait_gather(rows_a, sem_a)

            @pl.when(h > 0)
            def _():
                wait_out(acc_a, osem_a)

            compute_chunk(rows_a, acc_a, g0, s0, osem_a)

            @pl.when(h < SC_CHUNKS // 2 - 1)
            def _():
                start_gather(g0 + 2, rows_a, sem_a)

            wait_gather(rows_b, sem_b)

            @pl.when(h > 0)
            def _():
                wait_out(acc_b, osem_b)

            compute_chunk(rows_b, acc_b, g0 + 1, s0, osem_b)
            return carry2

        lax.fori_loop(0, SC_CHUNKS // 2, pair_body, 0)
        wait_out(acc_a, osem_a)
        wait_out(acc_b, osem_b)
        return carry

    lax.fori_loop(0, SPW // SS, super_body, 0)


def _feature_transform(ics_flat, vals_flat, table):
    mesh = plsc.VectorSubcoreMesh(core_axis_name="c", subcore_axis_name="s")
    return pl.kernel(
        _ft_body,
        out_type=jax.ShapeDtypeStruct((SAMPLES, D), jnp.float32),
        mesh=mesh,
        scratch_types=[
            pltpu.VMEM((SS * L,), jnp.int32),
            pltpu.VMEM((SS * L,), jnp.float32),
            pltpu.VMEM((CHUNK * L, D), jnp.float32),
            pltpu.VMEM((CHUNK * L, D), jnp.float32),
            pltpu.VMEM((CHUNK, D), jnp.float32),
            pltpu.VMEM((CHUNK, D), jnp.float32),
            pltpu.SemaphoreType.DMA,
            pltpu.SemaphoreType.DMA,
            pltpu.SemaphoreType.DMA,
            pltpu.SemaphoreType.DMA,
        ],
        name="nnue_feature_transform",
    )(ics_flat, vals_flat, table)


def _mlp_body(wf_ref, bf_ref, s_ref, bft_ref, W1_ref, b1_ref, W2_ref, b2_ref,
              Wo_ref, bo_ref, o_ref):
    bft = bft_ref[...]
    wf = wf_ref[...] + bft
    bf = bf_ref[...] + bft
    s = s_ref[...]
    x1 = jnp.clip((1.0 - s) * wf + s * bf, 0.0, 1.0)
    x2 = jnp.clip((1.0 - s) * bf + s * wf, 0.0, 1.0)
    dn = (((1,), (1,)), ((), ()))
    h = lax.dot_general(x1, W1_ref[:, :D], dn, preferred_element_type=jnp.float32)
    h += lax.dot_general(x2, W1_ref[:, D:], dn, preferred_element_type=jnp.float32)
    h = jnp.clip(h + b1_ref[...], 0.0, 1.0)
    h = lax.dot_general(h, W2_ref[...], dn, preferred_element_type=jnp.float32)
    h = jnp.clip(h + b2_ref[...], 0.0, 1.0)
    o_ref[...] = jnp.sum(h * Wo_ref[...], axis=1, keepdims=True) + bo_ref[...]


def _mlp_head(fts, stm, b_ft, W1, b1, W2, b2, Wo, bo):
    BB = 2048
    grid = (B // BB,)
    return pl.pallas_call(
        _mlp_body,
        grid=grid,
        in_specs=[
            pl.BlockSpec((BB, D), lambda i: (i, 0)),
            pl.BlockSpec((BB, D), lambda i: (B // BB + i, 0)),
            pl.BlockSpec((BB, 1), lambda i: (i, 0)),
            pl.BlockSpec((1, D), lambda i: (0, 0)),
            pl.BlockSpec((32, 2 * D), lambda i: (0, 0)),
            pl.BlockSpec((1, 32), lambda i: (0, 0)),
            pl.BlockSpec((32, 32), lambda i: (0, 0)),
            pl.BlockSpec((1, 32), lambda i: (0, 0)),
            pl.BlockSpec((1, 32), lambda i: (0, 0)),
            pl.BlockSpec((1, 1), lambda i: (0, 0)),
        ],
        out_specs=pl.BlockSpec((BB, 1), lambda i: (i, 0)),
        out_shape=jax.ShapeDtypeStruct((B, 1), jnp.float32),
    )(fts, fts, stm, b_ft, W1, b1, W2, b2, Wo, bo)


def kernel(wft_ics, wft_vals, bft_ics, bft_vals, stm, W_ft, b_ft, W1, b1, W2, b2, Wo, bo):
    ics_flat = jnp.concatenate([wft_ics, bft_ics]).reshape(-1)
    vals_flat = jnp.concatenate([wft_vals, bft_vals]).reshape(-1)
    fts = _feature_transform(ics_flat, vals_flat, W_ft)
    return _mlp_head(
        fts, stm,
        b_ft.reshape(1, D),
        W1, b1.reshape(1, 32), W2, b2.reshape(1, 32),
        Wo.reshape(1, 32), bo.reshape(1, 1),
    )
